# baseline (speedup 1.0000x reference)
import jax
import jax.numpy as jnp
from jax.experimental import pallas as pl


def _moddrop_body(x_ref, o_ref):
    xb = x_ref[...]
    ones_w = jnp.ones((xb.shape[3],), xb.dtype)
    ones_h = jnp.ones((xb.shape[2],), xb.dtype)
    t = jax.lax.dot_general(xb[0], ones_w, (((2,), (0,)), ((), ())),
                            preferred_element_type=jnp.float32)
    sums = jax.lax.dot_general(t, ones_h, (((1,), (0,)), ((), ())),
                               preferred_element_type=jnp.float32)
    gain = jnp.sum((sums != 0).astype(xb.dtype))
    o_ref[...] = xb * (1.0 / gain)


@jax.jit
def kernel(x):
    N, C, H, W = x.shape
    return pl.pallas_call(
        _moddrop_body,
        grid=(N,),
        in_specs=[pl.BlockSpec((1, C, H, W), lambda i: (i, 0, 0, 0))],
        out_specs=pl.BlockSpec((1, C, H, W), lambda i: (i, 0, 0, 0)),
        out_shape=jax.ShapeDtypeStruct(x.shape, x.dtype),
    )(x)


# speedup vs baseline: 1.0006x; 1.0006x over previous
import jax
import jax.numpy as jnp
from jax.experimental import pallas as pl


def _moddrop_body(x_ref, o_ref):
    xb = x_ref[...]
    sums = jnp.sum(xb, axis=(2, 3))
    gain = jnp.sum((sums != 0).astype(xb.dtype))
    o_ref[...] = xb / gain


@jax.jit
def kernel(x):
    N, C, H, W = x.shape
    return pl.pallas_call(
        _moddrop_body,
        grid=(N,),
        in_specs=[pl.BlockSpec((1, C, H, W), lambda i: (i, 0, 0, 0))],
        out_specs=pl.BlockSpec((1, C, H, W), lambda i: (i, 0, 0, 0)),
        out_shape=jax.ShapeDtypeStruct(x.shape, x.dtype),
    )(x)
